# baseline (speedup 1.0000x reference)
import functools

import jax
import jax.numpy as jnp
from jax import lax
from jax.experimental import pallas as pl
from jax.experimental.pallas import tpu as pltpu
from jax.experimental.pallas import tpu_sc as plsc

VOCAB = 1000000
EMB = 16
BATCH = 16384
HIST = 50
SEQ = HIST + 2

NC = 2
NS = 16
NW = NC * NS
BPW = BATCH // NW
CHB = 32
CH = CHB * HIST
NCH = BPW // CHB
NBUF = 3
NIDX = 2 * NBUF


@functools.lru_cache(maxsize=1)
def _build_sc_embed():
    mesh = plsc.VectorSubcoreMesh(core_axis_name="c", subcore_axis_name="s")

    @functools.partial(
        pl.kernel,
        mesh=mesh,
        out_type=jax.ShapeDtypeStruct((BATCH, SEQ, EMB), jnp.float32),
        compiler_params=pltpu.CompilerParams(use_tc_tiling_on_sc=False),
        scratch_types=[
            [pltpu.VMEM((CH,), jnp.int32) for _ in range(NIDX)],
            [pltpu.VMEM((CH, EMB), jnp.float32) for _ in range(NBUF)],
            [pltpu.VMEM((CHB, 2, EMB), jnp.float32) for _ in range(NBUF)],
            [pltpu.SemaphoreType.DMA for _ in range(NIDX)],
            [pltpu.SemaphoreType.DMA for _ in range(NBUF)],
            [pltpu.SemaphoreType.DMA for _ in range(NBUF)],
            [pltpu.SemaphoreType.DMA for _ in range(NBUF)],
            [pltpu.SemaphoreType.DMA for _ in range(NBUF)],
        ],
    )
    def _sc_embed(idx_h, rows_ui, t_h, out,
                  idx_bufs, row_bufs, ui_bufs,
                  isems, gsems, wsems, usems, uwsems):
        wid = lax.axis_index("s") * NC + lax.axis_index("c")
        bbase = wid * BPW

        def load_idx(c):
            slot = c % NIDX
            return pltpu.async_copy(
                idx_h.at[pl.ds((bbase + c * CHB) * HIST, CH)],
                idx_bufs[slot], isems[slot])

        def gather(c):
            return pltpu.async_copy(t_h.at[idx_bufs[c % NIDX]],
                                    row_bufs[c % NBUF], gsems[c % NBUF])

        def load_ui(c):
            slot = c % NBUF
            return pltpu.async_copy(rows_ui.at[pl.ds(bbase + c * CHB, CHB)],
                                    ui_bufs[slot], usems[slot])

        def write_out(c):
            slot = c % NBUF
            b0 = bbase + c * CHB

            def wbody(r, carry):
                pltpu.async_copy(
                    row_bufs[slot].at[pl.ds(r * HIST, HIST)],
                    out.at[b0 + r, pl.ds(0, HIST), :],
                    wsems[slot])
                return carry

            lax.fori_loop(0, CHB, wbody, 0)
            pltpu.async_copy(ui_bufs[slot],
                             out.at[pl.ds(b0, CHB), pl.ds(HIST, 2), :],
                             uwsems[slot])

        def drain_writes(c):
            slot = c % NBUF

            def dbody(r, carry):
                pltpu.make_async_copy(
                    row_bufs[slot].at[pl.ds(0, HIST)],
                    out.at[bbase, pl.ds(0, HIST), :],
                    wsems[slot]).wait()
                return carry

            lax.fori_loop(0, CHB, dbody, 0)
            pltpu.make_async_copy(ui_bufs[slot],
                                  out.at[pl.ds(bbase, CHB), pl.ds(HIST, 2), :],
                                  uwsems[slot]).wait()

        i_pend = {}
        u_pend = {}
        g_pend = {}
        w_open = []
        for c in range(min(NBUF, NCH)):
            i_pend[c % NIDX] = load_idx(c)
            u_pend[c % NBUF] = load_ui(c)

        for c in range(NCH):
            if c >= NBUF:
                drain_writes(c - NBUF)
                w_open.remove((c - NBUF) % NBUF)
                u_pend[c % NBUF] = load_ui(c)
            if c + NBUF < NCH:
                i_pend[(c + NBUF) % NIDX] = load_idx(c + NBUF)
            i_pend.pop(c % NIDX).wait()
            g_pend[c % NBUF] = gather(c)
            if c >= 1:
                p = (c - 1) % NBUF
                g_pend.pop(p).wait()
                u_pend.pop(p).wait()
                write_out(c - 1)
                w_open.append(p)

        p = (NCH - 1) % NBUF
        g_pend.pop(p).wait()
        u_pend.pop(p).wait()
        write_out(NCH - 1)
        w_open.append(p)
        for c_slot in list(w_open):
            drain_writes(c_slot)
            w_open.remove(c_slot)

    return _sc_embed


def kernel(hist_item, user_id, item_id, T_hist, T_user, T_item):
    idx_h = hist_item.astype(jnp.int32).reshape(-1)
    rows_u = jnp.take(T_user, user_id.reshape(-1), axis=0)
    rows_i = jnp.take(T_item, item_id.reshape(-1), axis=0)
    rows_ui = jnp.stack([rows_u, rows_i], axis=1)
    return _build_sc_embed()(idx_h, rows_ui, T_hist)
